# baseline (speedup 1.0000x reference)
import functools

import jax
import jax.numpy as jnp
from jax import lax
from jax.experimental import pallas as pl
from jax.experimental.pallas import tpu as pltpu
from jax.experimental.pallas import tpu_sc as plsc

_DEPTH = 2000
_N = 1024
_M = 50
_TOTAL = _N * _M * _DEPTH
_FILL_CHUNK = _DEPTH * _N

_NW = 32
_PER_W = (_N * _M) // _NW
_ROW = 64
_NROWS = _PER_W // _ROW


def _zero_block(out_ref):
    out_ref[...] = jnp.zeros_like(out_ref)


def _scatter_body(kflat_hbm, buf_hbm, idx_v, off_v, ones_v, sem):
    wid = lax.axis_index("s") * 2 + lax.axis_index("c")
    base = wid * _PER_W
    pltpu.sync_copy(kflat_hbm.at[pl.ds(base, _PER_W)], idx_v)
    for c in range(4):
        ones_v[pl.ds(c * 16, 16)] = jnp.full((16,), 1.0, jnp.float32)
    for c in range(_PER_W // 16):
        d = idx_v[pl.ds(c * 16, 16)]
        t = lax.broadcasted_iota(jnp.int32, (16,), 0) + (base + c * 16)
        j = lax.shift_right_logical(t, 10)
        i = jnp.bitwise_and(t, 1023)
        off = (
            j * (_DEPTH * _N)
            + lax.shift_left(lax.shift_right_logical(d, 3), 13)
            + lax.shift_left(lax.shift_right_logical(i, 7), 10)
            + lax.shift_left(jnp.bitwise_and(d, 7), 7)
            + jnp.bitwise_and(i, 127)
        )
        off_v[c // 4, pl.ds((c % 4) * 16, 16)] = off
    for r in range(_NROWS):
        pltpu.async_copy(ones_v, buf_hbm.at[off_v.at[r]], sem).start()
    for r in range(_NROWS):
        pltpu.async_copy(ones_v, buf_hbm.at[off_v.at[r]], sem).wait()


def _make_scatter():
    mesh = plsc.VectorSubcoreMesh(core_axis_name="c", subcore_axis_name="s")
    return pl.kernel(
        _scatter_body,
        out_type=(),
        mesh=mesh,
        scratch_types=[
            pltpu.VMEM((_PER_W,), jnp.int32),
            pltpu.VMEM((_NROWS, _ROW), jnp.int32),
            pltpu.VMEM((_ROW,), jnp.float32),
            pltpu.SemaphoreType.DMA,
        ],
    )


def kernel(inputs):
    n, m = inputs.shape
    kflat = inputs.astype(jnp.int32).T.reshape(n * m)
    zeros = pl.pallas_call(
        _zero_block,
        grid=(_TOTAL // _FILL_CHUNK,),
        in_specs=[],
        out_specs=pl.BlockSpec((_FILL_CHUNK,), lambda j: (j,)),
        out_shape=jax.ShapeDtypeStruct((_TOTAL,), jnp.float32),
    )()
    buf = jax.new_ref(zeros)
    _make_scatter()(kflat, buf)
    out = jax.freeze(buf)
    out = out.reshape(m, _DEPTH // 8, n // 128, 8, 128)
    out = out.transpose(2, 4, 0, 1, 3)
    return out.reshape(n, m, _DEPTH)


# speedup vs baseline: 2.3400x; 2.3400x over previous
import jax
import jax.numpy as jnp
from jax.experimental import pallas as pl

_DEPTH = 2000
_DBLK = 1000


def _onehot_block(idx_ref, out_ref):
    idx = idx_ref[0, 0, :]
    d0 = pl.program_id(1) * _DBLK
    iota = jax.lax.broadcasted_iota(jnp.int32, (_DBLK, idx.shape[0]), 0) + d0
    out_ref[0] = (iota == idx[None, :]).astype(jnp.float32)


def kernel(inputs):
    n, m = inputs.shape
    idx_t = inputs.astype(jnp.int32).T.reshape(m, 1, n)
    out = pl.pallas_call(
        _onehot_block,
        grid=(m, _DEPTH // _DBLK),
        in_specs=[pl.BlockSpec((1, 1, n), lambda j, k: (j, 0, 0))],
        out_specs=pl.BlockSpec((1, _DBLK, n), lambda j, k: (j, k, 0)),
        out_shape=jax.ShapeDtypeStruct((m, _DEPTH, n), jnp.float32),
    )(idx_t)
    return out.transpose(2, 0, 1)
